# Initial kernel scaffold; baseline (speedup 1.0000x reference)
import jax
import jax.numpy as jnp
from jax.experimental import pallas as pl


def kernel(x, table):
    raise NotImplementedError("write your pallas kernel here")


# speedup vs baseline: 10.1125x; 10.1125x over previous
import functools
import math

import jax
import jax.numpy as jnp
from jax import lax
from jax.experimental import pallas as pl
from jax.experimental.pallas import tpu as pltpu
from jax.experimental.pallas import tpu_sc as plsc

EMBED_DIM = 128
VOCAB = 22
SCALE = math.sqrt(float(EMBED_DIM))

NC = 2
NS = 16
NW = NC * NS
CHUNK = 128


@functools.partial(jax.jit, static_argnames=())
def _sc_embed(x3, table):
    nw, nchunks, chunk = x3.shape
    b_per_w = nchunks * chunk
    b_total = nw * b_per_w
    mesh = plsc.VectorSubcoreMesh(core_axis_name="c", subcore_axis_name="s")

    @functools.partial(
        pl.kernel,
        out_type=jax.ShapeDtypeStruct((b_total, EMBED_DIM), jnp.float32),
        mesh=mesh,
        scratch_types=[
            pltpu.VMEM((VOCAB, EMBED_DIM), jnp.float32),
            pltpu.VMEM_SHARED((VOCAB, EMBED_DIM), jnp.float32),
            pltpu.VMEM((nchunks, chunk), jnp.int32),
            pltpu.VMEM((chunk, EMBED_DIM), jnp.float32),
            pltpu.SemaphoreType.DMA,
            pltpu.SemaphoreType.DMA,
        ],
    )
    def k(x_hbm, table_hbm, out_hbm, table_v, table_sh, idx_v, buf, gsem, wsem):
        c = lax.axis_index("c")
        s = lax.axis_index("s")
        wid = s * NC + c
        base = wid * b_per_w

        @pl.when(s == 0)
        def _():
            pltpu.sync_copy(table_hbm, table_v)

            def scale_body(j, carry):
                r = j // (EMBED_DIM // 16)
                col = (j % (EMBED_DIM // 16)) * 16
                table_v[r, pl.ds(col, 16)] = table_v[r, pl.ds(col, 16)] * SCALE
                return carry

            lax.fori_loop(0, VOCAB * (EMBED_DIM // 16), scale_body, 0)
            pltpu.sync_copy(table_v, table_sh)

        plsc.subcore_barrier()

        pltpu.sync_copy(x_hbm.at[wid], idx_v)

        def body(g, carry):
            pltpu.async_copy(table_sh.at[idx_v.at[g]], buf, gsem).wait()
            pltpu.async_copy(
                buf, out_hbm.at[pl.ds(base + g * chunk, chunk)], wsem
            ).wait()
            return carry

        lax.fori_loop(0, nchunks, body, 0)

    return k(x3, table)


def kernel(x, table):
    rows, cols = x.shape
    b_total = rows * cols
    b_per_w = b_total // NW
    nchunks = b_per_w // CHUNK
    x3 = x.astype(jnp.int32).reshape(NW, nchunks, CHUNK)
    out = _sc_embed(x3, table)
    return out.reshape(rows, cols, EMBED_DIM)


# speedup vs baseline: 15.5747x; 1.5402x over previous
import functools
import math

import jax
import jax.numpy as jnp
from jax import lax
from jax.experimental import pallas as pl
from jax.experimental.pallas import tpu as pltpu
from jax.experimental.pallas import tpu_sc as plsc

EMBED_DIM = 128
VOCAB = 22
SCALE = math.sqrt(float(EMBED_DIM))

NC = 2
NS = 16
NW = NC * NS
CHUNK = 128


@functools.partial(jax.jit, static_argnames=())
def _sc_embed(x3, table):
    nw, nchunks, chunk = x3.shape
    b_per_w = nchunks * chunk
    b_total = nw * b_per_w
    mesh = plsc.VectorSubcoreMesh(core_axis_name="c", subcore_axis_name="s")

    @functools.partial(
        pl.kernel,
        out_type=jax.ShapeDtypeStruct((b_total, EMBED_DIM), jnp.float32),
        mesh=mesh,
        scratch_types=[
            pltpu.VMEM((VOCAB, EMBED_DIM), jnp.float32),
            pltpu.VMEM_SHARED((VOCAB, EMBED_DIM), jnp.float32),
            pltpu.VMEM((nchunks, chunk), jnp.int32),
            pltpu.VMEM((chunk, EMBED_DIM), jnp.float32),
            pltpu.VMEM((chunk, EMBED_DIM), jnp.float32),
            pltpu.SemaphoreType.DMA,
            pltpu.SemaphoreType.DMA,
            pltpu.SemaphoreType.DMA,
            pltpu.SemaphoreType.DMA,
        ],
    )
    def k(x_hbm, table_hbm, out_hbm, table_v, table_sh, idx_v,
          buf0, buf1, gsem0, gsem1, wsem0, wsem1):
        c = lax.axis_index("c")
        s = lax.axis_index("s")
        wid = s * NC + c
        base = wid * b_per_w

        @pl.when(s == 0)
        def _():
            pltpu.sync_copy(table_hbm, table_v)

            def scale_body(j, carry):
                r = j // (EMBED_DIM // 16)
                col = (j % (EMBED_DIM // 16)) * 16
                table_v[r, pl.ds(col, 16)] = table_v[r, pl.ds(col, 16)] * SCALE
                return carry

            lax.fori_loop(0, VOCAB * (EMBED_DIM // 16), scale_body, 0)
            pltpu.sync_copy(table_v, table_sh)

        plsc.subcore_barrier()

        pltpu.sync_copy(x_hbm.at[wid], idx_v)

        def out_at(g):
            return out_hbm.at[pl.ds(base + g * chunk, chunk)]

        pltpu.async_copy(table_sh.at[idx_v.at[0]], buf0, gsem0)
        ngroups = nchunks // 2

        def body(i, carry):
            g = 2 * i
            pltpu.make_async_copy(table_sh.at[idx_v.at[g]], buf0, gsem0).wait()

            @pl.when(i > 0)
            def _():
                pltpu.make_async_copy(buf1, out_at(g - 1), wsem1).wait()

            pltpu.async_copy(table_sh.at[idx_v.at[g + 1]], buf1, gsem1)
            pltpu.async_copy(buf0, out_at(g), wsem0)
            pltpu.make_async_copy(table_sh.at[idx_v.at[g + 1]], buf1, gsem1).wait()
            pltpu.make_async_copy(buf0, out_at(g), wsem0).wait()

            @pl.when(i + 1 < ngroups)
            def _():
                pltpu.async_copy(table_sh.at[idx_v.at[g + 2]], buf0, gsem0)

            pltpu.async_copy(buf1, out_at(g + 1), wsem1)
            return carry

        lax.fori_loop(0, ngroups, body, 0)
        pltpu.make_async_copy(buf1, out_at(nchunks - 1), wsem1).wait()

    return k(x3, table)


def kernel(x, table):
    rows, cols = x.shape
    b_total = rows * cols
    b_per_w = b_total // NW
    nchunks = b_per_w // CHUNK
    x3 = x.astype(jnp.int32).reshape(NW, nchunks, CHUNK)
    out = _sc_embed(x3, table)
    return out.reshape(rows, cols, EMBED_DIM)
